# baseline (speedup 1.0000x reference)
import functools

import jax
import jax.numpy as jnp
from jax import lax
from jax.experimental import pallas as pl
from jax.experimental.pallas import tpu as pltpu

_BLK = 2048
_NSLOT = 3


def _nn_body(x1_ref, x2_ref, corr_ref, idx12_ref, idx21_ref, minv_ref,
             buf, sems, *, kcols, blk):
    i = pl.program_id(0)
    nblocks = pl.num_programs(0)
    last_w = kcols - (nblocks - 1) * blk
    slot = lax.rem(i, _NSLOT)

    def full_copy(j, s):
        return pltpu.make_async_copy(
            buf.at[s], corr_ref.at[:, pl.ds(j * blk, blk)], sems.at[s])

    def tail_copy(j, s):
        return pltpu.make_async_copy(
            buf.at[s, :, :last_w],
            corr_ref.at[:, pl.ds(j * blk, last_w)], sems.at[s])

    @pl.when(i >= _NSLOT)
    def _():
        full_copy(i - _NSLOT, slot).wait()

    x1 = x1_ref[...]
    x2b = x2_ref[...]

    n1 = jnp.sum(x1 * x1, axis=-1)[:, None]
    n2 = jnp.sum(x2b * x2b, axis=-1)[None, :]
    prod = lax.dot_general(x1, x2b, (((1,), (1,)), ((), ())))
    d2 = n1 + n2 - 2.0 * prod
    dist = jnp.sqrt(jnp.maximum(d2, 0.0))
    buf[slot] = dist

    @pl.when(i < nblocks - 1)
    def _():
        full_copy(i, slot).start()

    @pl.when(i == nblocks - 1)
    def _():
        tail_copy(i, slot).start()

    def reduce_block(sub, width):
        idx12_ref[0, :width] = jnp.argmin(sub, axis=0)

        rmin = jnp.min(sub, axis=1, keepdims=True)
        rarg = jnp.argmin(sub, axis=1)[:, None] + i * blk

        @pl.when(i == 0)
        def _():
            minv_ref[...] = rmin
            idx21_ref[...] = rarg

        @pl.when(i > 0)
        def _():
            better = rmin < minv_ref[...]
            minv_ref[...] = jnp.where(better, rmin, minv_ref[...])
            idx21_ref[...] = jnp.where(better, rarg, idx21_ref[...])

    @pl.when(i < nblocks - 1)
    def _():
        reduce_block(dist, blk)

    @pl.when(i == nblocks - 1)
    def _():
        reduce_block(dist[:, :last_w], last_w)

    @pl.when(i == nblocks - 1)
    def _():
        full_copy(i - 2, lax.rem(i - 2, _NSLOT)).wait()
        full_copy(i - 1, lax.rem(i - 1, _NSLOT)).wait()
        tail_copy(i, slot).wait()


def kernel(x1, x2):
    n1, d = x1.shape
    n2, _ = x2.shape
    blk = _BLK
    kcols = (n2 // 128) * 128
    nblocks = pl.cdiv(kcols, blk)

    corr_k, idx12_k, idx21_k, minv_k = pl.pallas_call(
        functools.partial(_nn_body, kcols=kcols, blk=blk),
        grid=(nblocks,),
        in_specs=[
            pl.BlockSpec((n1, d), lambda i: (0, 0)),
            pl.BlockSpec((blk, d), lambda i: (i, 0)),
        ],
        out_specs=[
            pl.BlockSpec(memory_space=pl.ANY),
            pl.BlockSpec((1, blk), lambda i: (0, i)),
            pl.BlockSpec((n1, 1), lambda i: (0, 0)),
            pl.BlockSpec((n1, 1), lambda i: (0, 0)),
        ],
        out_shape=[
            jax.ShapeDtypeStruct((n1, n2), jnp.float32),
            jax.ShapeDtypeStruct((1, kcols), jnp.int32),
            jax.ShapeDtypeStruct((n1, 1), jnp.int32),
            jax.ShapeDtypeStruct((n1, 1), jnp.float32),
        ],
        scratch_shapes=[
            pltpu.VMEM((_NSLOT, n1, blk), jnp.float32),
            pltpu.SemaphoreType.DMA((_NSLOT,)),
        ],
    )(x1, x2)

    x2t = x2[kcols:]
    n1v = jnp.sum(x1 * x1, axis=-1)[:, None]
    n2t = jnp.sum(x2t * x2t, axis=-1)[None, :]
    d2t = n1v + n2t - 2.0 * jnp.matmul(x1, x2t.T)
    dt = jnp.sqrt(jnp.maximum(d2t, 0.0))

    corr = lax.dynamic_update_slice(corr_k, dt, (0, kcols))
    idx12 = jnp.concatenate([idx12_k[0], jnp.argmin(dt, axis=0)])
    tmin = jnp.min(dt, axis=1)
    targ = jnp.argmin(dt, axis=1) + kcols
    better = tmin < minv_k[:, 0]
    idx21 = jnp.where(better, targ, idx21_k[:, 0])
    return (x1, x2, corr, idx12, idx21)


# speedup vs baseline: 1.0102x; 1.0102x over previous
import functools

import jax
import jax.numpy as jnp
from jax import lax
from jax.experimental import pallas as pl
from jax.experimental.pallas import tpu as pltpu

_N1 = 1024
_D = 16
_BLK = 2048
_I32_MAX = jnp.iinfo(jnp.int32).max


def _nn_body(x1_ref, x2_ref, corr_ref, idx12_ref, idx21_ref, min_ref, *, n2_total, blk):
    i = pl.program_id(0)
    nblocks = pl.num_programs(0)
    last_w = n2_total - (n2_total // blk) * blk
    if last_w == 0:
        last_w = blk

    x1 = x1_ref[...]
    x2b = x2_ref[...]

    n1 = jnp.sum(x1 * x1, axis=-1)[:, None]
    n2 = jnp.sum(x2b * x2b, axis=-1)[None, :]
    prod = lax.dot_general(x1, x2b, (((1,), (1,)), ((), ())))
    d2 = n1 + n2 - 2.0 * prod
    dist = jnp.sqrt(jnp.maximum(d2, 0.0))
    corr_ref[...] = dist

    def reduce_block(sub2, sub, width):
        idx12_ref[0, :width] = jnp.argmin(sub2, axis=0)

        rmin = jnp.min(sub, axis=1, keepdims=True)
        rarg = jnp.argmin(sub, axis=1)[:, None] + i * blk

        @pl.when(i == 0)
        def _():
            min_ref[...] = rmin
            idx21_ref[...] = rarg

        @pl.when(i > 0)
        def _():
            better = rmin < min_ref[...]
            min_ref[...] = jnp.where(better, rmin, min_ref[...])
            idx21_ref[...] = jnp.where(better, rarg, idx21_ref[...])

    if last_w == blk:
        reduce_block(d2, dist, blk)
    else:
        @pl.when(i < nblocks - 1)
        def _():
            reduce_block(d2, dist, blk)

        @pl.when(i == nblocks - 1)
        def _():
            reduce_block(d2[:, :last_w], dist[:, :last_w], last_w)


def kernel(x1, x2):
    n1, d = x1.shape
    n2, _ = x2.shape
    blk = _BLK
    nblocks = pl.cdiv(n2, blk)

    corr, idx12, idx21 = pl.pallas_call(
        functools.partial(_nn_body, n2_total=n2, blk=blk),
        grid=(nblocks,),
        in_specs=[
            pl.BlockSpec((n1, d), lambda i: (0, 0)),
            pl.BlockSpec((blk, d), lambda i: (i, 0)),
        ],
        out_specs=[
            pl.BlockSpec((n1, blk), lambda i: (0, i)),
            pl.BlockSpec((1, blk), lambda i: (0, i)),
            pl.BlockSpec((n1, 1), lambda i: (0, 0)),
        ],
        out_shape=[
            jax.ShapeDtypeStruct((n1, n2), jnp.float32),
            jax.ShapeDtypeStruct((1, n2), jnp.int32),
            jax.ShapeDtypeStruct((n1, 1), jnp.int32),
        ],
        scratch_shapes=[pltpu.VMEM((n1, 1), jnp.float32)],
    )(x1, x2)

    return (x1, x2, corr, idx12[0], idx21[:, 0])


# speedup vs baseline: 1.0507x; 1.0400x over previous
import functools

import jax
import jax.numpy as jnp
from jax import lax
from jax.experimental import pallas as pl
from jax.experimental.pallas import tpu as pltpu

_N1 = 1024
_D = 16
_BLK = 4096
_I32_MAX = jnp.iinfo(jnp.int32).max


def _nn_body(x1_ref, x2_ref, corr_ref, idx12_ref, idx21_ref, min_ref, *, n2_total, blk):
    i = pl.program_id(0)
    nblocks = pl.num_programs(0)
    last_w = n2_total - (n2_total // blk) * blk
    if last_w == 0:
        last_w = blk

    x1 = x1_ref[...]
    x2b = x2_ref[...]

    n1 = jnp.sum(x1 * x1, axis=-1)[:, None]
    n2 = jnp.sum(x2b * x2b, axis=-1)[None, :]
    prod = lax.dot_general(x1, x2b, (((1,), (1,)), ((), ())))
    d2 = n1 + n2 - 2.0 * prod
    dist = jnp.sqrt(jnp.maximum(d2, 0.0))
    corr_ref[...] = dist

    def reduce_block(sub, width):
        idx12_ref[0, :width] = jnp.argmin(sub, axis=0)

        rmin = jnp.min(sub, axis=1, keepdims=True)
        rarg = jnp.argmin(sub, axis=1)[:, None] + i * blk

        @pl.when(i == 0)
        def _():
            min_ref[...] = rmin
            idx21_ref[...] = rarg

        @pl.when(i > 0)
        def _():
            better = rmin < min_ref[...]
            min_ref[...] = jnp.where(better, rmin, min_ref[...])
            idx21_ref[...] = jnp.where(better, rarg, idx21_ref[...])

    if last_w == blk:
        reduce_block(dist, blk)
    else:
        @pl.when(i < nblocks - 1)
        def _():
            reduce_block(dist, blk)

        @pl.when(i == nblocks - 1)
        def _():
            reduce_block(dist[:, :last_w], last_w)


def kernel(x1, x2):
    n1, d = x1.shape
    n2, _ = x2.shape
    blk = _BLK
    nblocks = pl.cdiv(n2, blk)

    corr, idx12, idx21 = pl.pallas_call(
        functools.partial(_nn_body, n2_total=n2, blk=blk),
        grid=(nblocks,),
        in_specs=[
            pl.BlockSpec((n1, d), lambda i: (0, 0)),
            pl.BlockSpec((blk, d), lambda i: (i, 0)),
        ],
        out_specs=[
            pl.BlockSpec((n1, blk), lambda i: (0, i)),
            pl.BlockSpec((1, blk), lambda i: (0, i)),
            pl.BlockSpec((n1, 1), lambda i: (0, 0)),
        ],
        out_shape=[
            jax.ShapeDtypeStruct((n1, n2), jnp.float32),
            jax.ShapeDtypeStruct((1, n2), jnp.int32),
            jax.ShapeDtypeStruct((n1, 1), jnp.int32),
        ],
        scratch_shapes=[pltpu.VMEM((n1, 1), jnp.float32)],
    )(x1, x2)

    return (x1, x2, corr, idx12[0], idx21[:, 0])
